# baseline (speedup 1.0000x reference)
import functools

import jax
import jax.numpy as jnp
import numpy as np
from jax import lax
from jax.experimental import pallas as pl
from jax.experimental.pallas import tpu as pltpu
from jax.experimental.pallas import tpu_sc as plsc

BATCH = 16384
EMB = 64
BN_EPS = 1e-5

NC = 2
NS = 16
NW = NC * NS
B_PER_W = BATCH // NW
CHUNK = 32
NCHUNK = B_PER_W // CHUNK
_D = np.arange(EMB)
_COLPERM = 32 * (_D // 32) + 2 * (_D % 16) + (_D % 32) // 16


def _sc_gmf(user, item, tug3, tig3):
    mesh = plsc.VectorSubcoreMesh(core_axis_name="c", subcore_axis_name="s")
    scratch = [
        pltpu.VMEM((B_PER_W,), jnp.int32),
        pltpu.VMEM((B_PER_W,), jnp.int32),
    ] + [pltpu.VMEM((CHUNK, EMB), jnp.float32)] * 6 + [
        pltpu.SemaphoreType.DMA,
        pltpu.SemaphoreType.DMA,
    ]

    @functools.partial(pl.kernel, mesh=mesh,
                       out_type=jax.ShapeDtypeStruct((BATCH, EMB),
                                                     jnp.float32),
                       scratch_types=scratch,
                       compiler_params=pltpu.CompilerParams(
                           needs_layout_passes=False))
    def k(u_hbm, i_hbm, tug, tig, o_gmf,
          idx_u, idx_i, fu0, fu1, fi0, fi1, p0, p1, sem0, sem1):
        wid = lax.axis_index("s") * NC + lax.axis_index("c")
        base = wid * B_PER_W
        pltpu.sync_copy(u_hbm.at[pl.ds(base, B_PER_W)], idx_u)
        pltpu.sync_copy(i_hbm.at[pl.ds(base, B_PER_W)], idx_i)
        fu = (fu0, fu1)
        fi = (fi0, fi1)
        pb = (p0, p1)
        sems = (sem0, sem1)

        def fire(c, slot):
            for g in range(CHUNK // 16):
                uu = idx_u[pl.ds(c * CHUNK + g * 16, 16)]
                vv = idx_i[pl.ds(c * CHUNK + g * 16, 16)]
                for l in range(16):
                    j = g * 16 + l
                    pltpu.async_copy(tug.at[uu[l] >> 4, uu[l] & 15],
                                     fu[slot].at[j], sems[slot])
                    pltpu.async_copy(tig.at[vv[l] >> 4, vv[l] & 15],
                                     fi[slot].at[j], sems[slot])

        def body2(t, carry):
            for k2 in range(2):
                c = 2 * t + k2
                slot = k2

                @pl.when(c + 1 < NCHUNK)
                def _():
                    fire(c + 1, 1 - k2)

                for _ in range(2 * CHUNK):
                    pltpu.make_async_copy(tug.at[0, 0], fu[slot].at[0],
                                          sems[slot]).wait()
                for l in range(CHUNK):
                    for q in range(EMB // 16):
                        cs = pl.ds(q * 16, 16)
                        pb[slot][l, cs] = fu[slot][l, cs] * fi[slot][l, cs]
                pltpu.sync_copy(
                    pb[slot], o_gmf.at[pl.ds(base + c * CHUNK, CHUNK)])
            return carry

        fire(0, 0)
        lax.fori_loop(0, NCHUNK // 2, body2, 0)

    return k(user, item, tug3, tig3)


def _sc_mlp(user, item, tum, tim):
    mesh = plsc.VectorSubcoreMesh(core_axis_name="c", subcore_axis_name="s")
    scratch = [
        pltpu.VMEM((B_PER_W,), jnp.int32),
        pltpu.VMEM((B_PER_W,), jnp.int32),
    ] + [pltpu.VMEM((2 * CHUNK, EMB), jnp.bfloat16)] * 4 + [
        pltpu.SemaphoreType.DMA,
        pltpu.SemaphoreType.DMA,
    ]

    @functools.partial(pl.kernel, mesh=mesh,
                       out_type=[jax.ShapeDtypeStruct((2 * BATCH, EMB),
                                                      jnp.bfloat16)] * 2,
                       scratch_types=scratch,
                       compiler_params=pltpu.CompilerParams(
                           needs_layout_passes=False))
    def k(u_hbm, i_hbm, tu, ti, o_um, o_im,
          idx_u, idx_i, bu0, bu1, bi0, bi1, sem0, sem1):
        wid = lax.axis_index("s") * NC + lax.axis_index("c")
        base = wid * B_PER_W
        pltpu.sync_copy(u_hbm.at[pl.ds(base, B_PER_W)], idx_u)
        pltpu.sync_copy(i_hbm.at[pl.ds(base, B_PER_W)], idx_i)
        bu = (bu0, bu1)
        bi = (bi0, bi1)
        sems = (sem0, sem1)

        def fire(c, slot):
            for g in range(CHUNK // 16):
                uu = idx_u[pl.ds(c * CHUNK + g * 16, 16)]
                vv = idx_i[pl.ds(c * CHUNK + g * 16, 16)]
                for l in range(16):
                    j = g * 16 + l
                    pltpu.async_copy(tu.at[pl.ds((uu[l] >> 1) * 2, 2)],
                                     bu[slot].at[pl.ds(2 * j, 2)],
                                     sems[slot])
                    pltpu.async_copy(ti.at[pl.ds((vv[l] >> 1) * 2, 2)],
                                     bi[slot].at[pl.ds(2 * j, 2)],
                                     sems[slot])

        def body2(t, carry):
            for k2 in range(2):
                c = 2 * t + k2
                slot = k2

                @pl.when(c + 1 < NCHUNK)
                def _():
                    fire(c + 1, 1 - k2)

                for _ in range(2 * CHUNK):
                    pltpu.make_async_copy(tu.at[pl.ds(0, 2)],
                                          bu[slot].at[pl.ds(0, 2)],
                                          sems[slot]).wait()
                dst = pl.ds(2 * (base + c * CHUNK), 2 * CHUNK)
                pltpu.sync_copy(bu[slot], o_um.at[dst])
                pltpu.sync_copy(bi[slot], o_im.at[dst])
            return carry

        fire(0, 0)
        lax.fori_loop(0, NCHUNK // 2, body2, 0)

    return k(user, item, tum, tim)


def _tc_dense(user, item, gmf, ump, imp, wfg, w0a, w0b, b0r,
              w1, b1r, w2, b2r, w3, b3r, wfm, consts):
    BB = 2048
    grid = BATCH // BB

    def sel(pair_ref, h):
        p = pair_ref[...].astype(jnp.float32).reshape(BB, 2, EMB)
        return p[:, 0, :] * (1.0 - h) + p[:, 1, :] * h

    def body(u_r, i_r, gmf_r, ump_r, imp_r, wfg_r, w0a_r, w0b_r,
             b0_r, w1_r, b1_r, w2_r, b2_r, w3_r, b3_r, wfm_r, c_r, out_r):
        f32 = jnp.float32
        hu = (u_r[0, :] & 1).astype(f32)[:, None]
        hi = (i_r[0, :] & 1).astype(f32)[:, None]
        um = sel(ump_r, hu)
        im = sel(imp_r, hi)
        x = jnp.dot(um, w0a_r[...], preferred_element_type=f32)
        x = x + jnp.dot(im, w0b_r[...], preferred_element_type=f32)
        x = jnp.maximum(x + b0_r[...], 0.0)
        x = jnp.maximum(jnp.dot(x, w1_r[...], preferred_element_type=f32)
                        + b1_r[...], 0.0)
        x = jnp.maximum(jnp.dot(x, w2_r[...], preferred_element_type=f32)
                        + b2_r[...], 0.0)
        x = jnp.maximum(jnp.dot(x, w3_r[...], preferred_element_type=f32)
                        + b3_r[...], 0.0)
        zg = jnp.sum(gmf_r[...] * wfg_r[...], axis=1)
        zm = jnp.sum(x * wfm_r[...], axis=1)
        z = zg + zm + c_r[0, 0]
        out_r[...] = c_r[0, 1] / (1.0 + jnp.exp(-z)) + c_r[0, 2]

    full = lambda shape: pl.BlockSpec(shape, lambda i: (0, 0))
    pair = lambda: pl.BlockSpec((2 * BB, EMB), lambda i: (i, 0))
    iv = lambda: pl.BlockSpec((1, BB), lambda i: (0, i))
    return pl.pallas_call(
        body,
        grid=(grid,),
        in_specs=[
            iv(), iv(), pl.BlockSpec((BB, EMB), lambda i: (i, 0)),
            pair(), pair(),
            full((1, EMB)),
            full((EMB, 128)), full((EMB, 128)), full((1, 128)),
            full((128, 128)), full((1, 128)),
            full((128, 128)), full((1, 128)),
            full((128, 128)), full((1, 128)),
            full((1, 128)), full((1, 128)),
        ],
        out_specs=pl.BlockSpec((BB,), lambda i: (i,)),
        out_shape=jax.ShapeDtypeStruct((BATCH,), jnp.float32),
    )(user.reshape(1, BATCH), item.reshape(1, BATCH), gmf, ump, imp,
      wfg, w0a, w0b, b0r, w1, b1r, w2, b2r, w3, b3r, wfm, consts)


def _pad2(a, r, c):
    return jnp.pad(a, ((0, r - a.shape[0]), (0, c - a.shape[1])))


def kernel(user, item, ue_gmf, ie_gmf, ue_mlp, ie_mlp,
           W0, b0, g0, beta0, W1, b1, g1, beta1,
           W2, b2, g2, beta2, W3, b3, g3, beta3,
           Wf, bf, scale, shift):
    ui = user.astype(jnp.int32)
    ii = item.astype(jnp.int32)
    gmf = _sc_gmf(ui, ii, ue_gmf.reshape(62500, 16, EMB),
                  ie_gmf.reshape(62500, 16, EMB))
    ump, imp = _sc_mlp(ui, ii, ue_mlp.astype(jnp.bfloat16),
                       ie_mlp.astype(jnp.bfloat16))

    inv = 1.0 / jnp.sqrt(jnp.float32(1.0 + BN_EPS))

    def fold(W, b, g, beta):
        s = inv * g
        return (W * s[:, None]).T, b * s + beta

    wt0, be0 = fold(W0, b0, g0, beta0)
    w0a, w0b = wt0[:EMB], wt0[EMB:]
    wt1, be1 = fold(W1, b1, g1, beta1)
    wt2, be2 = fold(W2, b2, g2, beta2)
    wt3, be3 = fold(W3, b3, g3, beta3)
    w1 = _pad2(wt1, 128, 128)
    w2 = _pad2(wt2, 128, 128)
    w3 = _pad2(wt3, 128, 128)
    b0r = be0.reshape(1, 128)
    b1r = _pad2(be1.reshape(1, -1), 1, 128)
    b2r = _pad2(be2.reshape(1, -1), 1, 128)
    b3r = _pad2(be3.reshape(1, -1), 1, 128)
    wfg = Wf[:, :EMB]
    wfm = _pad2(Wf[:, EMB:], 1, 128)
    consts = jnp.zeros((1, 128), jnp.float32)
    consts = consts.at[0, 0].set(bf[0]).at[0, 1].set(scale).at[0, 2].set(shift)

    return _tc_dense(ui, ii, gmf, ump, imp, wfg, w0a, w0b, b0r,
                     w1, b1r, w2, b2r, w3, b3r, wfm, consts)


# speedup vs baseline: 1.1136x; 1.1136x over previous
import functools

import jax
import jax.numpy as jnp
import numpy as np
from jax import lax
from jax.experimental import pallas as pl
from jax.experimental.pallas import tpu as pltpu
from jax.experimental.pallas import tpu_sc as plsc

BATCH = 16384
EMB = 64
BN_EPS = 1e-5

NC = 2
NS = 16
NW = NC * NS
B_PER_W = BATCH // NW
CHUNK = 32
NCHUNK = B_PER_W // CHUNK
_D = np.arange(EMB)
_COLPERM = 32 * (_D // 32) + 2 * (_D % 16) + (_D % 32) // 16


def _sc_one(idx, tab3):
    mesh = plsc.VectorSubcoreMesh(core_axis_name="c", subcore_axis_name="s")
    scratch = [
        pltpu.VMEM((B_PER_W,), jnp.int32),
        pltpu.VMEM((CHUNK, EMB), jnp.float32),
        pltpu.VMEM((CHUNK, EMB), jnp.float32),
        pltpu.SemaphoreType.DMA,
        pltpu.SemaphoreType.DMA,
    ]

    @functools.partial(pl.kernel, mesh=mesh,
                       out_type=jax.ShapeDtypeStruct((BATCH, EMB),
                                                     jnp.float32),
                       scratch_types=scratch,
                       compiler_params=pltpu.CompilerParams(
                           needs_layout_passes=False))
    def k(i_hbm, tab, out, idx_v, f0, f1, sem0, sem1):
        wid = lax.axis_index("s") * NC + lax.axis_index("c")
        base = wid * B_PER_W
        pltpu.sync_copy(i_hbm.at[pl.ds(base, B_PER_W)], idx_v)
        fb = (f0, f1)
        sems = (sem0, sem1)

        def fire(c, slot):
            for g in range(CHUNK // 16):
                uu = idx_v[pl.ds(c * CHUNK + g * 16, 16)]
                for l in range(16):
                    pltpu.async_copy(tab.at[uu[l] >> 4, uu[l] & 15],
                                     fb[slot].at[g * 16 + l], sems[slot])

        def body2(t, carry):
            for k2 in range(2):
                c = 2 * t + k2
                slot = k2

                @pl.when(c + 1 < NCHUNK)
                def _():
                    fire(c + 1, 1 - k2)

                for _ in range(CHUNK):
                    pltpu.make_async_copy(tab.at[0, 0], fb[slot].at[0],
                                          sems[slot]).wait()
                pltpu.sync_copy(
                    fb[slot], out.at[pl.ds(base + c * CHUNK, CHUNK)])
            return carry

        fire(0, 0)
        lax.fori_loop(0, NCHUNK // 2, body2, 0)

    return k(idx, tab3)


def _tc_dense(ug, ig, um, im, wfg, w0a, w0b, b0r,
              w1, b1r, w2, b2r, w3, b3r, wfm, consts):
    BB = 2048
    grid = BATCH // BB

    def body(ug_r, ig_r, um_r, im_r, wfg_r, w0a_r, w0b_r,
             b0_r, w1_r, b1_r, w2_r, b2_r, w3_r, b3_r, wfm_r, c_r, out_r):
        f32 = jnp.float32
        x = jnp.dot(um_r[...], w0a_r[...], preferred_element_type=f32)
        x = x + jnp.dot(im_r[...], w0b_r[...], preferred_element_type=f32)
        x = jnp.maximum(x + b0_r[...], 0.0)
        x = jnp.maximum(jnp.dot(x, w1_r[...], preferred_element_type=f32)
                        + b1_r[...], 0.0)
        x = jnp.maximum(jnp.dot(x, w2_r[...], preferred_element_type=f32)
                        + b2_r[...], 0.0)
        x = jnp.maximum(jnp.dot(x, w3_r[...], preferred_element_type=f32)
                        + b3_r[...], 0.0)
        zg = jnp.sum(ug_r[...] * ig_r[...] * wfg_r[...], axis=1)
        zm = jnp.sum(x * wfm_r[...], axis=1)
        z = zg + zm + c_r[0, 0]
        out_r[...] = c_r[0, 1] / (1.0 + jnp.exp(-z)) + c_r[0, 2]

    full = lambda shape: pl.BlockSpec(shape, lambda i: (0, 0))
    row = lambda: pl.BlockSpec((BB, EMB), lambda i: (i, 0))
    return pl.pallas_call(
        body,
        grid=(grid,),
        in_specs=[
            row(), row(), row(), row(),
            full((1, EMB)),
            full((EMB, 128)), full((EMB, 128)), full((1, 128)),
            full((128, 128)), full((1, 128)),
            full((128, 128)), full((1, 128)),
            full((128, 128)), full((1, 128)),
            full((1, 128)), full((1, 128)),
        ],
        out_specs=pl.BlockSpec((BB,), lambda i: (i,)),
        out_shape=jax.ShapeDtypeStruct((BATCH,), jnp.float32),
    )(ug, ig, um, im, wfg, w0a, w0b, b0r, w1, b1r, w2, b2r, w3, b3r,
      wfm, consts)


def _pad2(a, r, c):
    return jnp.pad(a, ((0, r - a.shape[0]), (0, c - a.shape[1])))


def kernel(user, item, ue_gmf, ie_gmf, ue_mlp, ie_mlp,
           W0, b0, g0, beta0, W1, b1, g1, beta1,
           W2, b2, g2, beta2, W3, b3, g3, beta3,
           Wf, bf, scale, shift):
    ui = user.astype(jnp.int32)
    ii = item.astype(jnp.int32)
    ug = _sc_one(ui, ue_gmf.reshape(62500, 16, EMB))
    ig = _sc_one(ii, ie_gmf.reshape(62500, 16, EMB))
    um = _sc_one(ui, ue_mlp.reshape(62500, 16, EMB))
    im = _sc_one(ii, ie_mlp.reshape(62500, 16, EMB))

    inv = 1.0 / jnp.sqrt(jnp.float32(1.0 + BN_EPS))

    def fold(W, b, g, beta):
        s = inv * g
        return (W * s[:, None]).T, b * s + beta

    wt0, be0 = fold(W0, b0, g0, beta0)
    w0a, w0b = wt0[:EMB], wt0[EMB:]
    wt1, be1 = fold(W1, b1, g1, beta1)
    wt2, be2 = fold(W2, b2, g2, beta2)
    wt3, be3 = fold(W3, b3, g3, beta3)
    w1 = _pad2(wt1, 128, 128)
    w2 = _pad2(wt2, 128, 128)
    w3 = _pad2(wt3, 128, 128)
    b0r = be0.reshape(1, 128)
    b1r = _pad2(be1.reshape(1, -1), 1, 128)
    b2r = _pad2(be2.reshape(1, -1), 1, 128)
    b3r = _pad2(be3.reshape(1, -1), 1, 128)
    wfg = Wf[:, :EMB]
    wfm = _pad2(Wf[:, EMB:], 1, 128)
    consts = jnp.zeros((1, 128), jnp.float32)
    consts = consts.at[0, 0].set(bf[0]).at[0, 1].set(scale).at[0, 2].set(shift)

    return _tc_dense(ug, ig, um, im, wfg, w0a, w0b, b0r,
                     w1, b1r, w2, b2r, w3, b3r, wfm, consts)


# speedup vs baseline: 1.1497x; 1.0324x over previous
import functools

import jax
import jax.numpy as jnp
import numpy as np
from jax import lax
from jax.experimental import pallas as pl
from jax.experimental.pallas import tpu as pltpu
from jax.experimental.pallas import tpu_sc as plsc

BATCH = 16384
EMB = 64
BN_EPS = 1e-5

NC = 2
NS = 16
NW = NC * NS
B_PER_W = BATCH // NW
CHUNK = 32
NCHUNK = B_PER_W // CHUNK
_D = np.arange(EMB)
_COLPERM = 32 * (_D // 32) + 2 * (_D % 16) + (_D % 32) // 16


def _sc_one(idx, tab3):
    mesh = plsc.VectorSubcoreMesh(core_axis_name="c", subcore_axis_name="s")
    scratch = [
        pltpu.VMEM((B_PER_W,), jnp.int32),
        pltpu.VMEM((CHUNK, EMB), jnp.float32),
        pltpu.VMEM((CHUNK, EMB), jnp.float32),
        pltpu.SemaphoreType.DMA,
        pltpu.SemaphoreType.DMA,
    ]

    @functools.partial(pl.kernel, mesh=mesh,
                       out_type=jax.ShapeDtypeStruct((BATCH, EMB),
                                                     jnp.float32),
                       scratch_types=scratch,
                       compiler_params=pltpu.CompilerParams(
                           needs_layout_passes=False))
    def k(i_hbm, tab, out, idx_v, f0, f1, sem0, sem1):
        wid = lax.axis_index("s") * NC + lax.axis_index("c")
        base = wid * B_PER_W
        pltpu.sync_copy(i_hbm.at[pl.ds(base, B_PER_W)], idx_v)
        fb = (f0, f1)
        sems = (sem0, sem1)

        def fire(c, slot):
            for g in range(CHUNK // 16):
                uu = idx_v[pl.ds(c * CHUNK + g * 16, 16)]
                for l in range(16):
                    pltpu.async_copy(tab.at[uu[l] >> 4, uu[l] & 15],
                                     fb[slot].at[g * 16 + l], sems[slot])

        def body2(t, carry):
            for k2 in range(2):
                c = 2 * t + k2
                slot = k2

                @pl.when(c + 1 < NCHUNK)
                def _():
                    fire(c + 1, 1 - k2)

                for _ in range(CHUNK):
                    pltpu.make_async_copy(tab.at[0, 0], fb[slot].at[0],
                                          sems[slot]).wait()
                pltpu.sync_copy(
                    fb[slot], out.at[pl.ds(base + c * CHUNK, CHUNK)])
            return carry

        fire(0, 0)
        lax.fori_loop(0, NCHUNK // 2, body2, 0)

    return k(idx, tab3)


def _sc_pair(idx, tab):
    mesh = plsc.VectorSubcoreMesh(core_axis_name="c", subcore_axis_name="s")
    scratch = [
        pltpu.VMEM((B_PER_W,), jnp.int32),
        pltpu.VMEM((2 * CHUNK, EMB), jnp.bfloat16),
        pltpu.VMEM((2 * CHUNK, EMB), jnp.bfloat16),
        pltpu.SemaphoreType.DMA,
        pltpu.SemaphoreType.DMA,
    ]

    @functools.partial(pl.kernel, mesh=mesh,
                       out_type=jax.ShapeDtypeStruct((2 * BATCH, EMB),
                                                     jnp.bfloat16),
                       scratch_types=scratch,
                       compiler_params=pltpu.CompilerParams(
                           needs_layout_passes=False))
    def k(i_hbm, tb, out, idx_v, b0, b1, sem0, sem1):
        wid = lax.axis_index("s") * NC + lax.axis_index("c")
        base = wid * B_PER_W
        pltpu.sync_copy(i_hbm.at[pl.ds(base, B_PER_W)], idx_v)
        bb = (b0, b1)
        sems = (sem0, sem1)

        def fire(c, slot):
            for g in range(CHUNK // 16):
                uu = idx_v[pl.ds(c * CHUNK + g * 16, 16)]
                for l in range(16):
                    j = g * 16 + l
                    pltpu.async_copy(tb.at[pl.ds((uu[l] >> 1) * 2, 2)],
                                     bb[slot].at[pl.ds(2 * j, 2)],
                                     sems[slot])

        def body2(t, carry):
            for k2 in range(2):
                c = 2 * t + k2
                slot = k2

                @pl.when(c + 1 < NCHUNK)
                def _():
                    fire(c + 1, 1 - k2)

                for _ in range(CHUNK):
                    pltpu.make_async_copy(tb.at[pl.ds(0, 2)],
                                          bb[slot].at[pl.ds(0, 2)],
                                          sems[slot]).wait()
                pltpu.sync_copy(
                    bb[slot],
                    out.at[pl.ds(2 * (base + c * CHUNK), 2 * CHUNK)])
            return carry

        fire(0, 0)
        lax.fori_loop(0, NCHUNK // 2, body2, 0)

    return k(idx, tab)


def _tc_dense(item, ug, ig, um, imp, wfg, w0a, w0b, b0r,
              w1, b1r, w2, b2r, w3, b3r, wfm, consts):
    BB = 2048
    grid = BATCH // BB

    def body(i_r, ug_r, ig_r, um_r, imp_r, wfg_r, w0a_r, w0b_r,
             b0_r, w1_r, b1_r, w2_r, b2_r, w3_r, b3_r, wfm_r, c_r, out_r):
        f32 = jnp.float32
        hi = (i_r[0, :] & 1).astype(f32)[:, None]
        p = imp_r[...].astype(f32).reshape(BB, 2, EMB)
        im = p[:, 0, :] * (1.0 - hi) + p[:, 1, :] * hi
        x = jnp.dot(um_r[...], w0a_r[...], preferred_element_type=f32)
        x = x + jnp.dot(im, w0b_r[...], preferred_element_type=f32)
        x = jnp.maximum(x + b0_r[...], 0.0)
        x = jnp.maximum(jnp.dot(x, w1_r[...], preferred_element_type=f32)
                        + b1_r[...], 0.0)
        x = jnp.maximum(jnp.dot(x, w2_r[...], preferred_element_type=f32)
                        + b2_r[...], 0.0)
        x = jnp.maximum(jnp.dot(x, w3_r[...], preferred_element_type=f32)
                        + b3_r[...], 0.0)
        zg = jnp.sum(ug_r[...] * ig_r[...] * wfg_r[...], axis=1)
        zm = jnp.sum(x * wfm_r[...], axis=1)
        z = zg + zm + c_r[0, 0]
        out_r[...] = c_r[0, 1] / (1.0 + jnp.exp(-z)) + c_r[0, 2]

    full = lambda shape: pl.BlockSpec(shape, lambda i: (0, 0))
    row = lambda: pl.BlockSpec((BB, EMB), lambda i: (i, 0))
    return pl.pallas_call(
        body,
        grid=(grid,),
        in_specs=[
            pl.BlockSpec((1, BB), lambda i: (0, i)),
            row(), row(), row(),
            pl.BlockSpec((2 * BB, EMB), lambda i: (i, 0)),
            full((1, EMB)),
            full((EMB, 128)), full((EMB, 128)), full((1, 128)),
            full((128, 128)), full((1, 128)),
            full((128, 128)), full((1, 128)),
            full((128, 128)), full((1, 128)),
            full((1, 128)), full((1, 128)),
        ],
        out_specs=pl.BlockSpec((BB,), lambda i: (i,)),
        out_shape=jax.ShapeDtypeStruct((BATCH,), jnp.float32),
    )(item.reshape(1, BATCH), ug, ig, um, imp, wfg, w0a, w0b, b0r,
      w1, b1r, w2, b2r, w3, b3r, wfm, consts)


def _pad2(a, r, c):
    return jnp.pad(a, ((0, r - a.shape[0]), (0, c - a.shape[1])))


def kernel(user, item, ue_gmf, ie_gmf, ue_mlp, ie_mlp,
           W0, b0, g0, beta0, W1, b1, g1, beta1,
           W2, b2, g2, beta2, W3, b3, g3, beta3,
           Wf, bf, scale, shift):
    ui = user.astype(jnp.int32)
    ii = item.astype(jnp.int32)
    ug = _sc_one(ui, ue_gmf.reshape(62500, 16, EMB))
    ig = _sc_one(ii, ie_gmf.reshape(62500, 16, EMB))
    um = _sc_one(ui, ue_mlp.reshape(62500, 16, EMB))
    imp = _sc_pair(ii, ie_mlp.astype(jnp.bfloat16))

    inv = 1.0 / jnp.sqrt(jnp.float32(1.0 + BN_EPS))

    def fold(W, b, g, beta):
        s = inv * g
        return (W * s[:, None]).T, b * s + beta

    wt0, be0 = fold(W0, b0, g0, beta0)
    w0a, w0b = wt0[:EMB], wt0[EMB:]
    wt1, be1 = fold(W1, b1, g1, beta1)
    wt2, be2 = fold(W2, b2, g2, beta2)
    wt3, be3 = fold(W3, b3, g3, beta3)
    w1 = _pad2(wt1, 128, 128)
    w2 = _pad2(wt2, 128, 128)
    w3 = _pad2(wt3, 128, 128)
    b0r = be0.reshape(1, 128)
    b1r = _pad2(be1.reshape(1, -1), 1, 128)
    b2r = _pad2(be2.reshape(1, -1), 1, 128)
    b3r = _pad2(be3.reshape(1, -1), 1, 128)
    wfg = Wf[:, :EMB]
    wfm = _pad2(Wf[:, EMB:], 1, 128)
    consts = jnp.zeros((1, 128), jnp.float32)
    consts = consts.at[0, 0].set(bf[0]).at[0, 1].set(scale).at[0, 2].set(shift)

    return _tc_dense(ii, ug, ig, um, imp, wfg, w0a, w0b, b0r,
                     w1, b1r, w2, b2r, w3, b3r, wfm, consts)


# speedup vs baseline: 1.1626x; 1.0112x over previous
import functools

import jax
import jax.numpy as jnp
import numpy as np
from jax import lax
from jax.experimental import pallas as pl
from jax.experimental.pallas import tpu as pltpu
from jax.experimental.pallas import tpu_sc as plsc

BATCH = 16384
EMB = 64
BN_EPS = 1e-5

NC = 2
NS = 16
NW = NC * NS
B_PER_W = BATCH // NW
CHUNK = 32
NCHUNK = B_PER_W // CHUNK
_D = np.arange(EMB)
_COLPERM = 32 * (_D // 32) + 2 * (_D % 16) + (_D % 32) // 16


def _sc_one(idx, tab3):
    mesh = plsc.VectorSubcoreMesh(core_axis_name="c", subcore_axis_name="s")
    scratch = [
        pltpu.VMEM((B_PER_W,), jnp.int32),
        pltpu.VMEM((CHUNK, EMB), jnp.float32),
        pltpu.VMEM((CHUNK, EMB), jnp.float32),
        pltpu.SemaphoreType.DMA,
        pltpu.SemaphoreType.DMA,
    ]

    @functools.partial(pl.kernel, mesh=mesh,
                       out_type=jax.ShapeDtypeStruct((BATCH, EMB),
                                                     jnp.float32),
                       scratch_types=scratch,
                       compiler_params=pltpu.CompilerParams(
                           needs_layout_passes=False))
    def k(i_hbm, tab, out, idx_v, f0, f1, sem0, sem1):
        wid = lax.axis_index("s") * NC + lax.axis_index("c")
        base = wid * B_PER_W
        pltpu.sync_copy(i_hbm.at[pl.ds(base, B_PER_W)], idx_v)
        fb = (f0, f1)
        sems = (sem0, sem1)

        def fire(c, slot):
            for g in range(CHUNK // 16):
                uu = idx_v[pl.ds(c * CHUNK + g * 16, 16)]
                for l in range(16):
                    pltpu.async_copy(tab.at[uu[l] >> 4, uu[l] & 15],
                                     fb[slot].at[g * 16 + l], sems[slot])

        def body2(t, carry):
            for k2 in range(2):
                c = 2 * t + k2
                slot = k2

                @pl.when(c + 1 < NCHUNK)
                def _():
                    fire(c + 1, 1 - k2)

                for _ in range(CHUNK):
                    pltpu.make_async_copy(tab.at[0, 0], fb[slot].at[0],
                                          sems[slot]).wait()
                pltpu.sync_copy(
                    fb[slot], out.at[pl.ds(base + c * CHUNK, CHUNK)])
            return carry

        fire(0, 0)
        lax.fori_loop(0, NCHUNK // 2, body2, 0)

    return k(idx, tab3)


def _sc_trip(user, item, tug3, tig3, tum3):
    mesh = plsc.VectorSubcoreMesh(core_axis_name="c", subcore_axis_name="s")
    scratch = [
        pltpu.VMEM((B_PER_W,), jnp.int32),
        pltpu.VMEM((B_PER_W,), jnp.int32),
    ] + [pltpu.VMEM((CHUNK, EMB), jnp.float32)] * 8 + [
        pltpu.SemaphoreType.DMA,
        pltpu.SemaphoreType.DMA,
    ]

    @functools.partial(pl.kernel, mesh=mesh,
                       out_type=[jax.ShapeDtypeStruct((BATCH, EMB),
                                                      jnp.float32)] * 2,
                       scratch_types=scratch,
                       compiler_params=pltpu.CompilerParams(
                           needs_layout_passes=False))
    def k(u_hbm, i_hbm, tug, tig, tum, o_gmf, o_um,
          idx_u, idx_i, fu0, fu1, fi0, fi1, fm0, fm1, p0, p1, sem0, sem1):
        wid = lax.axis_index("s") * NC + lax.axis_index("c")
        base = wid * B_PER_W
        pltpu.sync_copy(u_hbm.at[pl.ds(base, B_PER_W)], idx_u)
        pltpu.sync_copy(i_hbm.at[pl.ds(base, B_PER_W)], idx_i)
        fu = (fu0, fu1)
        fi = (fi0, fi1)
        fm = (fm0, fm1)
        pb = (p0, p1)
        sems = (sem0, sem1)

        def fire(c, slot):
            for g in range(CHUNK // 16):
                uu = idx_u[pl.ds(c * CHUNK + g * 16, 16)]
                vv = idx_i[pl.ds(c * CHUNK + g * 16, 16)]
                for l in range(16):
                    j = g * 16 + l
                    pltpu.async_copy(tug.at[uu[l] >> 4, uu[l] & 15],
                                     fu[slot].at[j], sems[slot])
                    pltpu.async_copy(tum.at[uu[l] >> 4, uu[l] & 15],
                                     fm[slot].at[j], sems[slot])
                    pltpu.async_copy(tig.at[vv[l] >> 4, vv[l] & 15],
                                     fi[slot].at[j], sems[slot])

        def body2(t, carry):
            for k2 in range(2):
                c = 2 * t + k2
                slot = k2

                @pl.when(c + 1 < NCHUNK)
                def _():
                    fire(c + 1, 1 - k2)

                for _ in range(3 * CHUNK):
                    pltpu.make_async_copy(tug.at[0, 0], fu[slot].at[0],
                                          sems[slot]).wait()
                for l in range(CHUNK):
                    for q in range(EMB // 16):
                        cs = pl.ds(q * 16, 16)
                        pb[slot][l, cs] = fu[slot][l, cs] * fi[slot][l, cs]
                sl = pl.ds(base + c * CHUNK, CHUNK)
                pltpu.sync_copy(pb[slot], o_gmf.at[sl])
                pltpu.sync_copy(fm[slot], o_um.at[sl])
            return carry

        fire(0, 0)
        lax.fori_loop(0, NCHUNK // 2, body2, 0)

    return k(user, item, tug3, tig3, tum3)


def _sc_pair(idx, tab):
    mesh = plsc.VectorSubcoreMesh(core_axis_name="c", subcore_axis_name="s")
    scratch = [
        pltpu.VMEM((B_PER_W,), jnp.int32),
        pltpu.VMEM((2 * CHUNK, EMB), jnp.bfloat16),
        pltpu.VMEM((2 * CHUNK, EMB), jnp.bfloat16),
        pltpu.SemaphoreType.DMA,
        pltpu.SemaphoreType.DMA,
    ]

    @functools.partial(pl.kernel, mesh=mesh,
                       out_type=jax.ShapeDtypeStruct((2 * BATCH, EMB),
                                                     jnp.bfloat16),
                       scratch_types=scratch,
                       compiler_params=pltpu.CompilerParams(
                           needs_layout_passes=False))
    def k(i_hbm, tb, out, idx_v, b0, b1, sem0, sem1):
        wid = lax.axis_index("s") * NC + lax.axis_index("c")
        base = wid * B_PER_W
        pltpu.sync_copy(i_hbm.at[pl.ds(base, B_PER_W)], idx_v)
        bb = (b0, b1)
        sems = (sem0, sem1)

        def fire(c, slot):
            for g in range(CHUNK // 16):
                uu = idx_v[pl.ds(c * CHUNK + g * 16, 16)]
                for l in range(16):
                    j = g * 16 + l
                    pltpu.async_copy(tb.at[pl.ds((uu[l] >> 1) * 2, 2)],
                                     bb[slot].at[pl.ds(2 * j, 2)],
                                     sems[slot])

        def body2(t, carry):
            for k2 in range(2):
                c = 2 * t + k2
                slot = k2

                @pl.when(c + 1 < NCHUNK)
                def _():
                    fire(c + 1, 1 - k2)

                for _ in range(CHUNK):
                    pltpu.make_async_copy(tb.at[pl.ds(0, 2)],
                                          bb[slot].at[pl.ds(0, 2)],
                                          sems[slot]).wait()
                pltpu.sync_copy(
                    bb[slot],
                    out.at[pl.ds(2 * (base + c * CHUNK), 2 * CHUNK)])
            return carry

        fire(0, 0)
        lax.fori_loop(0, NCHUNK // 2, body2, 0)

    return k(idx, tab)


def _tc_dense(item, gmf, um, imp, wfg, w0a, w0b, b0r,
              w1, b1r, w2, b2r, w3, b3r, wfm, consts):
    BB = 2048
    grid = BATCH // BB

    def body(i_r, gmf_r, um_r, imp_r, wfg_r, w0a_r, w0b_r,
             b0_r, w1_r, b1_r, w2_r, b2_r, w3_r, b3_r, wfm_r, c_r, out_r):
        f32 = jnp.float32
        hi = (i_r[0, :] & 1).astype(f32)[:, None]
        p = imp_r[...].astype(f32).reshape(BB, 2, EMB)
        im = p[:, 0, :] * (1.0 - hi) + p[:, 1, :] * hi
        x = jnp.dot(um_r[...], w0a_r[...], preferred_element_type=f32)
        x = x + jnp.dot(im, w0b_r[...], preferred_element_type=f32)
        x = jnp.maximum(x + b0_r[...], 0.0)
        x = jnp.maximum(jnp.dot(x, w1_r[...], preferred_element_type=f32)
                        + b1_r[...], 0.0)
        x = jnp.maximum(jnp.dot(x, w2_r[...], preferred_element_type=f32)
                        + b2_r[...], 0.0)
        x = jnp.maximum(jnp.dot(x, w3_r[...], preferred_element_type=f32)
                        + b3_r[...], 0.0)
        zg = jnp.sum(gmf_r[...] * wfg_r[...], axis=1)
        zm = jnp.sum(x * wfm_r[...], axis=1)
        z = zg + zm + c_r[0, 0]
        out_r[...] = c_r[0, 1] / (1.0 + jnp.exp(-z)) + c_r[0, 2]

    full = lambda shape: pl.BlockSpec(shape, lambda i: (0, 0))
    row = lambda: pl.BlockSpec((BB, EMB), lambda i: (i, 0))
    return pl.pallas_call(
        body,
        grid=(grid,),
        in_specs=[
            pl.BlockSpec((1, BB), lambda i: (0, i)),
            row(), row(),
            pl.BlockSpec((2 * BB, EMB), lambda i: (i, 0)),
            full((1, EMB)),
            full((EMB, 128)), full((EMB, 128)), full((1, 128)),
            full((128, 128)), full((1, 128)),
            full((128, 128)), full((1, 128)),
            full((128, 128)), full((1, 128)),
            full((1, 128)), full((1, 128)),
        ],
        out_specs=pl.BlockSpec((BB,), lambda i: (i,)),
        out_shape=jax.ShapeDtypeStruct((BATCH,), jnp.float32),
    )(item.reshape(1, BATCH), gmf, um, imp, wfg, w0a, w0b, b0r,
      w1, b1r, w2, b2r, w3, b3r, wfm, consts)


def _pad2(a, r, c):
    return jnp.pad(a, ((0, r - a.shape[0]), (0, c - a.shape[1])))


def kernel(user, item, ue_gmf, ie_gmf, ue_mlp, ie_mlp,
           W0, b0, g0, beta0, W1, b1, g1, beta1,
           W2, b2, g2, beta2, W3, b3, g3, beta3,
           Wf, bf, scale, shift):
    ui = user.astype(jnp.int32)
    ii = item.astype(jnp.int32)
    gmf, um = _sc_trip(ui, ii, ue_gmf.reshape(62500, 16, EMB),
                       ie_gmf.reshape(62500, 16, EMB),
                       ue_mlp.reshape(62500, 16, EMB))
    imp = _sc_pair(ii, ie_mlp.astype(jnp.bfloat16))

    inv = 1.0 / jnp.sqrt(jnp.float32(1.0 + BN_EPS))

    def fold(W, b, g, beta):
        s = inv * g
        return (W * s[:, None]).T, b * s + beta

    wt0, be0 = fold(W0, b0, g0, beta0)
    w0a, w0b = wt0[:EMB], wt0[EMB:]
    wt1, be1 = fold(W1, b1, g1, beta1)
    wt2, be2 = fold(W2, b2, g2, beta2)
    wt3, be3 = fold(W3, b3, g3, beta3)
    w1 = _pad2(wt1, 128, 128)
    w2 = _pad2(wt2, 128, 128)
    w3 = _pad2(wt3, 128, 128)
    b0r = be0.reshape(1, 128)
    b1r = _pad2(be1.reshape(1, -1), 1, 128)
    b2r = _pad2(be2.reshape(1, -1), 1, 128)
    b3r = _pad2(be3.reshape(1, -1), 1, 128)
    wfg = Wf[:, :EMB]
    wfm = _pad2(Wf[:, EMB:], 1, 128)
    consts = jnp.zeros((1, 128), jnp.float32)
    consts = consts.at[0, 0].set(bf[0]).at[0, 1].set(scale).at[0, 2].set(shift)

    return _tc_dense(ii, gmf, um, imp, wfg, w0a, w0b, b0r,
                     w1, b1r, w2, b2r, w3, b3r, wfm, consts)
